# baseline (speedup 1.0000x reference)
import functools

import jax
import jax.numpy as jnp
from jax import lax
from jax.experimental import pallas as pl
from jax.experimental.pallas import tpu as pltpu
from jax.experimental.pallas import tpu_sc as plsc

_NMEANS = 2000
_NKNOT = _NMEANS + 4
_NPAD = 2048
_L = 16


def _tdigest_cdf_body(x_hbm, means_hbm, weights_hbm, out_hbm,
                      e_v, c_v, stage_m, cumw_s, w_v,
                      xbuf_a, obuf_a, xbuf_b, obuf_b,
                      lv32, lv16, lv8, lv4, lv2, lv1,
                      isem_a, osem_a, isem_b, osem_b,
                      *, n_total, n_workers, chunk):
    wid = lax.axis_index("s") * 2 + lax.axis_index("c")
    per_w = n_total // n_workers
    base = wid * per_w

    pltpu.sync_copy(means_hbm, stage_m)
    pltpu.sync_copy(weights_hbm, w_v)

    iota = lax.iota(jnp.int32, _L)
    lane15 = jnp.full((_L,), _L - 1, jnp.int32)
    _dnums = lax.GatherDimensionNumbers(
        offset_dims=(), collapsed_slice_dims=(0,), start_index_map=(0,))

    def _perm(v, idx):
        return lax.gather(v, idx[:, None], _dnums, slice_sizes=(1,),
                          mode=lax.GatherScatterMode.PROMISE_IN_BOUNDS)

    def _cumsum16(v):
        for sh in (1, 2, 4, 8):
            g = _perm(v, jnp.maximum(iota - sh, 0))
            v = v + jnp.where(iota >= sh, g, 0.0)
        return v

    def cum_body(i, carry):
        wv = w_v[pl.ds(i * _L, _L)]
        c = _cumsum16(wv) + carry
        cumw_s[pl.ds(i * _L, _L)] = c - wv * 0.5
        return _perm(c, lane15)

    total_v = lax.fori_loop(0, _NMEANS // _L, cum_body,
                            jnp.zeros((_L,), jnp.float32))
    inv_w = 1.0 / total_v

    zero_i = jnp.zeros((_L,), jnp.int32)
    m0 = plsc.load_gather(stage_m, [zero_i])
    mn = plsc.load_gather(stage_m, [zero_i + (_NMEANS - 1)])
    mean_min = m0 - 1.0
    mean_max = mn + 1.0

    mn_bits = plsc.bitcast(mn, jnp.int32)
    nb_bits = jnp.where(mn > 0.0, mn_bits - 1,
                        jnp.where(mn < 0.0, mn_bits + 1,
                                  jnp.full((_L,), -2**31, jnp.int32)))
    nb = plsc.bitcast(nb_bits, jnp.float32)
    e_first = jnp.minimum(mean_min, nb)
    inf_v = jnp.full((_L,), jnp.inf, jnp.float32)
    zero_f = jnp.zeros((_L,), jnp.float32)

    def knot_body(k, _):
        posv = k * _L + iota
        srcc = jnp.clip(posv - 2, 0, _NMEANS - 1)
        g = plsc.load_gather(stage_m, [srcc])
        e = jnp.minimum(g, nb)
        e = jnp.where(posv == 0, e_first, e)
        e = jnp.where(posv == _NKNOT - 1, mean_max, e)
        e = jnp.where(posv >= _NKNOT, inf_v, e)
        e_v[pl.ds(k * _L, _L)] = e
        q = plsc.load_gather(cumw_s, [srcc])
        q = jnp.where(posv <= 1, zero_f, q)
        q = jnp.where(posv >= _NKNOT - 2, total_v, q)
        c_v[pl.ds(k * _L, _L)] = q
        return 0

    lax.fori_loop(0, _NPAD // _L, knot_body, 0)

    t1 = plsc.load_gather(e_v, [iota * 128 + 127])
    t2 = plsc.load_gather(e_v, [iota * 128 + 63])

    lv_tables = ((lv32, 32, 6), (lv16, 16, 5), (lv8, 8, 4),
                 (lv4, 4, 3), (lv2, 2, 2), (lv1, 1, 1))
    for arr, s, _sh in lv_tables:
        n_lv = _NPAD // (2 * s)

        def lv_body(k, _, arr=arr, s=s):
            idxv = (k * _L + iota) * (2 * s) + (s - 1)
            arr[pl.ds(k * _L, _L)] = plsc.load_gather(e_v, [idxv])
            return 0

        lax.fori_loop(0, n_lv // _L, lv_body, 0)

    def make_compute(xbuf, obuf):
      def compute_vreg(j):
        xv = xbuf[pl.ds(j * _L, _L)]
        lo4 = zero_i
        for step in (8, 4, 2, 1):
            g = _perm(t1, lo4 + (step - 1))
            lo4 = jnp.where(g < xv, lo4 + step, lo4)
        g5 = _perm(t2, lo4)
        lo = lo4 * 128 + jnp.where(g5 < xv, 64, 0)
        for arr, s, sh in lv_tables:
            probe = plsc.load_gather(arr, [lo >> sh])
            lo = jnp.where(probe < xv, lo + s, lo)
        u = jnp.clip(lo, 1, _NKNOT - 1)
        u1 = u - 1
        e1 = plsc.load_gather(e_v, [u1])
        e2 = plsc.load_gather(e_v, [u])
        c1 = plsc.load_gather(c_v, [u1])
        c2 = plsc.load_gather(c_v, [u])
        xc = jnp.clip(xv, e1, e2)
        z1 = xc - e1
        z2 = e2 - xc
        den = z1 + z2
        safe = jnp.where(den == 0.0, 1.0, den)
        wa = (c1 * z1 + c2 * z2) / safe
        obuf[pl.ds(j * _L, _L)] = wa * inv_w
      return compute_vreg

    bufs = ((xbuf_a, obuf_a, isem_a, osem_a),
            (xbuf_b, obuf_b, isem_b, osem_b))
    nch = per_w // chunk

    def _in_copy(g, xb, sem):
        return pltpu.make_async_copy(
            x_hbm.at[pl.ds(base + g * chunk, chunk)], xb, sem)

    def _out_copy(g, ob, sem):
        return pltpu.make_async_copy(
            ob, out_hbm.at[pl.ds(base + g * chunk, chunk)], sem)

    _in_copy(0, xbuf_a, isem_a).start()

    def pair_body(i, _):
        for b in range(2):
            g = 2 * i + b
            xb, ob, isem, osem = bufs[b]
            xb2, _ob2, isem2, _osem2 = bufs[1 - b]

            @pl.when(g + 1 < nch)
            def _():
                _in_copy(g + 1, xb2, isem2).start()

            _in_copy(g, xb, isem).wait()

            @pl.when(g >= 2)
            def _():
                _out_copy(g - 2, ob, osem).wait()

            plsc.parallel_loop(0, chunk // _L, unroll=48)(
                make_compute(xb, ob))
            _out_copy(g, ob, osem).start()
        return 0

    lax.fori_loop(0, nch // 2, pair_body, 0)
    _out_copy(nch - 2, obuf_a, osem_a).wait()
    _out_copy(nch - 1, obuf_b, osem_b).wait()


def kernel(x, processed_means, processed_weights):
    n_total = x.shape[0]
    info = plsc.get_sparse_core_info()
    n_workers = info.num_cores * info.num_subcores
    chunk = 16384
    mesh = plsc.VectorSubcoreMesh(core_axis_name="c", subcore_axis_name="s")
    body = functools.partial(_tdigest_cdf_body, n_total=n_total,
                             n_workers=n_workers, chunk=chunk)
    fn = pl.kernel(
        body,
        out_type=jax.ShapeDtypeStruct((n_total,), jnp.float32),
        mesh=mesh,
        compiler_params=pltpu.CompilerParams(needs_layout_passes=False),
        scratch_types=[
            pltpu.VMEM((_NPAD,), jnp.float32),
            pltpu.VMEM((_NPAD,), jnp.float32),
            pltpu.VMEM((_NMEANS,), jnp.float32),
            pltpu.VMEM((_NMEANS,), jnp.float32),
            pltpu.VMEM((_NMEANS,), jnp.float32),
            pltpu.VMEM((chunk,), jnp.float32),
            pltpu.VMEM((chunk,), jnp.float32),
            pltpu.VMEM((chunk,), jnp.float32),
            pltpu.VMEM((chunk,), jnp.float32),
            pltpu.VMEM((32,), jnp.float32),
            pltpu.VMEM((64,), jnp.float32),
            pltpu.VMEM((128,), jnp.float32),
            pltpu.VMEM((256,), jnp.float32),
            pltpu.VMEM((512,), jnp.float32),
            pltpu.VMEM((1024,), jnp.float32),
            pltpu.SemaphoreType.DMA,
            pltpu.SemaphoreType.DMA,
            pltpu.SemaphoreType.DMA,
            pltpu.SemaphoreType.DMA,
        ],
    )
    return fn(x, processed_means.astype(jnp.float32),
              processed_weights.astype(jnp.float32))


# speedup vs baseline: 1.0953x; 1.0953x over previous
import functools

import jax
import jax.numpy as jnp
from jax import lax
from jax.experimental import pallas as pl
from jax.experimental.pallas import tpu as pltpu
from jax.experimental.pallas import tpu_sc as plsc

_NMEANS = 2000
_NKNOT = _NMEANS + 4
_NPAD = 2048
_L = 16


def _tdigest_cdf_body(x_hbm, means_hbm, weights_hbm, out_hbm,
                      e_v, c_v, stage_m, cumw_s, w_v,
                      xbuf_a, obuf_a, xbuf_b, obuf_b,
                      lv32, lv16, lv8, lv4, lv2, lv1,
                      isem_a, osem_a, isem_b, osem_b,
                      *, n_total, n_workers, chunk):
    wid = lax.axis_index("s") * 2 + lax.axis_index("c")
    per_w = n_total // n_workers
    base = wid * per_w

    pltpu.sync_copy(means_hbm, stage_m)
    pltpu.sync_copy(weights_hbm, w_v)

    iota = lax.iota(jnp.int32, _L)
    lane15 = jnp.full((_L,), _L - 1, jnp.int32)
    _dnums = lax.GatherDimensionNumbers(
        offset_dims=(), collapsed_slice_dims=(0,), start_index_map=(0,))

    def _perm(v, idx):
        return lax.gather(v, idx[:, None], _dnums, slice_sizes=(1,),
                          mode=lax.GatherScatterMode.PROMISE_IN_BOUNDS)

    def _cumsum16(v):
        for sh in (1, 2, 4, 8):
            g = _perm(v, jnp.maximum(iota - sh, 0))
            v = v + jnp.where(iota >= sh, g, 0.0)
        return v

    def cum_body(i, carry):
        wv = w_v[pl.ds(i * _L, _L)]
        c = _cumsum16(wv) + carry
        cumw_s[pl.ds(i * _L, _L)] = c - wv * 0.5
        return _perm(c, lane15)

    total_v = lax.fori_loop(0, _NMEANS // _L, cum_body,
                            jnp.zeros((_L,), jnp.float32))
    inv_w = 1.0 / total_v

    zero_i = jnp.zeros((_L,), jnp.int32)
    m0 = plsc.load_gather(stage_m, [zero_i])
    mn = plsc.load_gather(stage_m, [zero_i + (_NMEANS - 1)])
    mean_min = m0 - 1.0
    mean_max = mn + 1.0

    mn_bits = plsc.bitcast(mn, jnp.int32)
    nb_bits = jnp.where(mn > 0.0, mn_bits - 1,
                        jnp.where(mn < 0.0, mn_bits + 1,
                                  jnp.full((_L,), -2**31, jnp.int32)))
    nb = plsc.bitcast(nb_bits, jnp.float32)
    e_first = jnp.minimum(mean_min, nb)
    inf_v = jnp.full((_L,), jnp.inf, jnp.float32)
    zero_f = jnp.zeros((_L,), jnp.float32)

    def knot_body(k, _):
        posv = k * _L + iota
        srcc = jnp.clip(posv - 2, 0, _NMEANS - 1)
        g = plsc.load_gather(stage_m, [srcc])
        e = jnp.minimum(g, nb)
        e = jnp.where(posv == 0, e_first, e)
        e = jnp.where(posv == _NKNOT - 1, mean_max, e)
        e = jnp.where(posv >= _NKNOT, inf_v, e)
        e_v[pl.ds(k * _L, _L)] = e
        q = plsc.load_gather(cumw_s, [srcc])
        q = jnp.where(posv <= 1, zero_f, q)
        q = jnp.where(posv >= _NKNOT - 2, total_v, q)
        c_v[pl.ds(k * _L, _L)] = q
        return 0

    lax.fori_loop(0, _NPAD // _L, knot_body, 0)

    t1 = plsc.load_gather(e_v, [iota * 128 + 127])
    t2 = plsc.load_gather(e_v, [iota * 128 + 63])
    p7 = plsc.load_gather(e_v, [zero_i + 1023])
    p3 = plsc.load_gather(e_v, [zero_i + 511])
    p11 = plsc.load_gather(e_v, [zero_i + 1535])

    lv_tables = ((lv32, 32, 6), (lv16, 16, 5), (lv8, 8, 4),
                 (lv4, 4, 3), (lv2, 2, 2), (lv1, 1, 1))
    for arr, s, _sh in lv_tables:
        n_lv = _NPAD // (2 * s)

        def lv_body(k, _, arr=arr, s=s):
            idxv = (k * _L + iota) * (2 * s) + (s - 1)
            arr[pl.ds(k * _L, _L)] = plsc.load_gather(e_v, [idxv])
            return 0

        lax.fori_loop(0, n_lv // _L, lv_body, 0)

    def make_compute(xbuf, obuf):
      def compute_vreg(j):
        xv = xbuf[pl.ds(j * _L, _L)]
        c1 = p7 < xv
        lo4 = jnp.where(c1, 8, 0)
        g2 = jnp.where(c1, p11, p3)
        lo4 = jnp.where(g2 < xv, lo4 + 4, lo4)
        for step in (2, 1):
            g = _perm(t1, lo4 + (step - 1))
            lo4 = jnp.where(g < xv, lo4 + step, lo4)
        g5 = _perm(t2, lo4)
        lo = lo4 * 128 + jnp.where(g5 < xv, 64, 0)
        for arr, s, sh in lv_tables:
            probe = plsc.load_gather(arr, [lo >> sh])
            lo = jnp.where(probe < xv, lo + s, lo)
        u = jnp.clip(lo, 1, _NKNOT - 1)
        u1 = u - 1
        e1 = plsc.load_gather(e_v, [u1])
        e2 = plsc.load_gather(e_v, [u])
        c1 = plsc.load_gather(c_v, [u1])
        c2 = plsc.load_gather(c_v, [u])
        xc = jnp.clip(xv, e1, e2)
        z1 = xc - e1
        z2 = e2 - xc
        den = z1 + z2
        safe = jnp.where(den == 0.0, 1.0, den)
        wa = (c1 * z1 + c2 * z2) / safe
        obuf[pl.ds(j * _L, _L)] = wa * inv_w
      return compute_vreg

    bufs = ((xbuf_a, obuf_a, isem_a, osem_a),
            (xbuf_b, obuf_b, isem_b, osem_b))
    nch = per_w // chunk

    def _in_copy(g, xb, sem):
        return pltpu.make_async_copy(
            x_hbm.at[pl.ds(base + g * chunk, chunk)], xb, sem)

    def _out_copy(g, ob, sem):
        return pltpu.make_async_copy(
            ob, out_hbm.at[pl.ds(base + g * chunk, chunk)], sem)

    _in_copy(0, xbuf_a, isem_a).start()

    def pair_body(i, _):
        for b in range(2):
            g = 2 * i + b
            xb, ob, isem, osem = bufs[b]
            xb2, _ob2, isem2, _osem2 = bufs[1 - b]

            @pl.when(g + 1 < nch)
            def _():
                _in_copy(g + 1, xb2, isem2).start()

            _in_copy(g, xb, isem).wait()

            @pl.when(g >= 2)
            def _():
                _out_copy(g - 2, ob, osem).wait()

            plsc.parallel_loop(0, chunk // _L, unroll=32)(
                make_compute(xb, ob))
            _out_copy(g, ob, osem).start()
        return 0

    lax.fori_loop(0, nch // 2, pair_body, 0)
    _out_copy(nch - 2, obuf_a, osem_a).wait()
    _out_copy(nch - 1, obuf_b, osem_b).wait()


def kernel(x, processed_means, processed_weights):
    n_total = x.shape[0]
    info = plsc.get_sparse_core_info()
    n_workers = info.num_cores * info.num_subcores
    chunk = 16384
    mesh = plsc.VectorSubcoreMesh(core_axis_name="c", subcore_axis_name="s")
    body = functools.partial(_tdigest_cdf_body, n_total=n_total,
                             n_workers=n_workers, chunk=chunk)
    fn = pl.kernel(
        body,
        out_type=jax.ShapeDtypeStruct((n_total,), jnp.float32),
        mesh=mesh,
        compiler_params=pltpu.CompilerParams(needs_layout_passes=False),
        scratch_types=[
            pltpu.VMEM((_NPAD,), jnp.float32),
            pltpu.VMEM((_NPAD,), jnp.float32),
            pltpu.VMEM((_NMEANS,), jnp.float32),
            pltpu.VMEM((_NMEANS,), jnp.float32),
            pltpu.VMEM((_NMEANS,), jnp.float32),
            pltpu.VMEM((chunk,), jnp.float32),
            pltpu.VMEM((chunk,), jnp.float32),
            pltpu.VMEM((chunk,), jnp.float32),
            pltpu.VMEM((chunk,), jnp.float32),
            pltpu.VMEM((32,), jnp.float32),
            pltpu.VMEM((64,), jnp.float32),
            pltpu.VMEM((128,), jnp.float32),
            pltpu.VMEM((256,), jnp.float32),
            pltpu.VMEM((512,), jnp.float32),
            pltpu.VMEM((1024,), jnp.float32),
            pltpu.SemaphoreType.DMA,
            pltpu.SemaphoreType.DMA,
            pltpu.SemaphoreType.DMA,
            pltpu.SemaphoreType.DMA,
        ],
    )
    return fn(x, processed_means.astype(jnp.float32),
              processed_weights.astype(jnp.float32))


# speedup vs baseline: 1.1108x; 1.0141x over previous
import functools

import jax
import jax.numpy as jnp
from jax import lax
from jax.experimental import pallas as pl
from jax.experimental.pallas import tpu as pltpu
from jax.experimental.pallas import tpu_sc as plsc

_NMEANS = 2000
_NKNOT = _NMEANS + 4
_NPAD = 2048
_L = 16


def _tdigest_cdf_body(x_hbm, means_hbm, weights_hbm, out_hbm,
                      e_v, c_v, stage_m, cumw_s, w_v,
                      xbuf_a, obuf_a, xbuf_b, obuf_b,
                      lv32, lv16, lv8, lv4, lv2, lv1,
                      isem_a, osem_a, isem_b, osem_b,
                      *, n_total, n_workers, chunk):
    wid = lax.axis_index("s") * 2 + lax.axis_index("c")
    per_w = n_total // n_workers
    base = wid * per_w

    pltpu.sync_copy(means_hbm, stage_m)
    pltpu.sync_copy(weights_hbm, w_v)

    iota = lax.iota(jnp.int32, _L)
    lane15 = jnp.full((_L,), _L - 1, jnp.int32)
    _dnums = lax.GatherDimensionNumbers(
        offset_dims=(), collapsed_slice_dims=(0,), start_index_map=(0,))

    def _perm(v, idx):
        return lax.gather(v, idx[:, None], _dnums, slice_sizes=(1,),
                          mode=lax.GatherScatterMode.PROMISE_IN_BOUNDS)

    def _cumsum16(v):
        for sh in (1, 2, 4, 8):
            g = _perm(v, jnp.maximum(iota - sh, 0))
            v = v + jnp.where(iota >= sh, g, 0.0)
        return v

    def cum_body(i, carry):
        wv = w_v[pl.ds(i * _L, _L)]
        c = _cumsum16(wv) + carry
        cumw_s[pl.ds(i * _L, _L)] = c - wv * 0.5
        return _perm(c, lane15)

    total_v = lax.fori_loop(0, _NMEANS // _L, cum_body,
                            jnp.zeros((_L,), jnp.float32))
    inv_w = 1.0 / total_v

    zero_i = jnp.zeros((_L,), jnp.int32)
    m0 = plsc.load_gather(stage_m, [zero_i])
    mn = plsc.load_gather(stage_m, [zero_i + (_NMEANS - 1)])
    mean_min = m0 - 1.0
    mean_max = mn + 1.0

    mn_bits = plsc.bitcast(mn, jnp.int32)
    nb_bits = jnp.where(mn > 0.0, mn_bits - 1,
                        jnp.where(mn < 0.0, mn_bits + 1,
                                  jnp.full((_L,), -2**31, jnp.int32)))
    nb = plsc.bitcast(nb_bits, jnp.float32)
    e_first = jnp.minimum(mean_min, nb)
    inf_v = jnp.full((_L,), jnp.inf, jnp.float32)
    zero_f = jnp.zeros((_L,), jnp.float32)

    def knot_body(k, _):
        posv = k * _L + iota
        srcc = jnp.clip(posv - 2, 0, _NMEANS - 1)
        g = plsc.load_gather(stage_m, [srcc])
        e = jnp.minimum(g, nb)
        e = jnp.where(posv == 0, e_first, e)
        e = jnp.where(posv == _NKNOT - 1, mean_max, e)
        e = jnp.where(posv >= _NKNOT, inf_v, e)
        e_v[pl.ds(k * _L, _L)] = e
        q = plsc.load_gather(cumw_s, [srcc])
        q = jnp.where(posv <= 1, zero_f, q)
        q = jnp.where(posv >= _NKNOT - 2, total_v, q)
        c_v[pl.ds(k * _L, _L)] = q
        return 0

    lax.fori_loop(0, _NPAD // _L, knot_body, 0)

    t1 = plsc.load_gather(e_v, [iota * 128 + 127])
    t2 = plsc.load_gather(e_v, [iota * 128 + 63])

    lv_tables = ((lv32, 32, 6), (lv16, 16, 5), (lv8, 8, 4),
                 (lv4, 4, 3), (lv2, 2, 2), (lv1, 1, 1))
    for arr, s, _sh in lv_tables:
        n_lv = _NPAD // (2 * s)

        def lv_body(k, _, arr=arr, s=s):
            idxv = (k * _L + iota) * (2 * s) + (s - 1)
            arr[pl.ds(k * _L, _L)] = plsc.load_gather(e_v, [idxv])
            return 0

        lax.fori_loop(0, n_lv // _L, lv_body, 0)

    def make_compute(xbuf, obuf):
      def compute_vreg(j):
        xv = xbuf[pl.ds(j * _L, _L)]
        lo4 = zero_i
        for step in (8, 4, 2, 1):
            g = _perm(t1, lo4 + (step - 1))
            lo4 = jnp.where(g < xv, lo4 + step, lo4)
        g5 = _perm(t2, lo4)
        lo = lo4 * 128 + jnp.where(g5 < xv, 64, 0)
        for arr, s, sh in lv_tables:
            probe = plsc.load_gather(arr, [lo >> sh])
            lo = jnp.where(probe < xv, lo + s, lo)
        u = jnp.clip(lo, 1, _NKNOT - 1)
        u1 = u - 1
        e1 = plsc.load_gather(e_v, [u1])
        e2 = plsc.load_gather(e_v, [u])
        c1 = plsc.load_gather(c_v, [u1])
        c2 = plsc.load_gather(c_v, [u])
        xc = jnp.clip(xv, e1, e2)
        z1 = xc - e1
        z2 = e2 - xc
        den = z1 + z2
        safe = jnp.where(den == 0.0, 1.0, den)
        wa = (c1 * z1 + c2 * z2) / safe
        obuf[pl.ds(j * _L, _L)] = wa * inv_w
      return compute_vreg

    bufs = ((xbuf_a, obuf_a, isem_a, osem_a),
            (xbuf_b, obuf_b, isem_b, osem_b))
    nch = per_w // chunk

    def _in_copy(g, xb, sem):
        return pltpu.make_async_copy(
            x_hbm.at[pl.ds(base + g * chunk, chunk)], xb, sem)

    def _out_copy(g, ob, sem):
        return pltpu.make_async_copy(
            ob, out_hbm.at[pl.ds(base + g * chunk, chunk)], sem)

    _in_copy(0, xbuf_a, isem_a).start()

    def pair_body(i, _):
        for b in range(2):
            g = 2 * i + b
            xb, ob, isem, osem = bufs[b]
            xb2, _ob2, isem2, _osem2 = bufs[1 - b]

            @pl.when(g + 1 < nch)
            def _():
                _in_copy(g + 1, xb2, isem2).start()

            _in_copy(g, xb, isem).wait()

            @pl.when(g >= 2)
            def _():
                _out_copy(g - 2, ob, osem).wait()

            plsc.parallel_loop(0, chunk // _L, unroll=32)(
                make_compute(xb, ob))
            _out_copy(g, ob, osem).start()
        return 0

    lax.fori_loop(0, nch // 2, pair_body, 0)
    _out_copy(nch - 2, obuf_a, osem_a).wait()
    _out_copy(nch - 1, obuf_b, osem_b).wait()


def kernel(x, processed_means, processed_weights):
    n_total = x.shape[0]
    info = plsc.get_sparse_core_info()
    n_workers = info.num_cores * info.num_subcores
    chunk = 16384
    mesh = plsc.VectorSubcoreMesh(core_axis_name="c", subcore_axis_name="s")
    body = functools.partial(_tdigest_cdf_body, n_total=n_total,
                             n_workers=n_workers, chunk=chunk)
    fn = pl.kernel(
        body,
        out_type=jax.ShapeDtypeStruct((n_total,), jnp.float32),
        mesh=mesh,
        compiler_params=pltpu.CompilerParams(needs_layout_passes=False),
        scratch_types=[
            pltpu.VMEM((_NPAD,), jnp.float32),
            pltpu.VMEM((_NPAD,), jnp.float32),
            pltpu.VMEM((_NMEANS,), jnp.float32),
            pltpu.VMEM((_NMEANS,), jnp.float32),
            pltpu.VMEM((_NMEANS,), jnp.float32),
            pltpu.VMEM((chunk,), jnp.float32),
            pltpu.VMEM((chunk,), jnp.float32),
            pltpu.VMEM((chunk,), jnp.float32),
            pltpu.VMEM((chunk,), jnp.float32),
            pltpu.VMEM((32,), jnp.float32),
            pltpu.VMEM((64,), jnp.float32),
            pltpu.VMEM((128,), jnp.float32),
            pltpu.VMEM((256,), jnp.float32),
            pltpu.VMEM((512,), jnp.float32),
            pltpu.VMEM((1024,), jnp.float32),
            pltpu.SemaphoreType.DMA,
            pltpu.SemaphoreType.DMA,
            pltpu.SemaphoreType.DMA,
            pltpu.SemaphoreType.DMA,
        ],
    )
    return fn(x, processed_means.astype(jnp.float32),
              processed_weights.astype(jnp.float32))
